# Initial kernel scaffold; baseline (speedup 1.0000x reference)
import jax
import jax.numpy as jnp
from jax.experimental import pallas as pl


def kernel(x_User, x_Computer, x_Group, x_Domain, x_CertTemplate, x_Resource, x_AIAgent, ei_0, ei_1, ei_2, ei_3, ei_4, ei_5, ei_6, ei_7, theta, params):
    raise NotImplementedError("write your pallas kernel here")


# speedup vs baseline: 1.1553x; 1.1553x over previous
import functools

import jax
import jax.numpy as jnp
from jax import lax
from jax.experimental import pallas as pl
from jax.experimental.pallas import tpu as pltpu
from jax.experimental.pallas import tpu_sc as plsc

H = 128
HEADS = 4
HD = H // HEADS
_NC, _NS = 2, 16
_NW = _NC * _NS
_CHUNK = 64
_EALIGN = _NW * _CHUNK

_NTS = ['User', 'Computer', 'Group', 'Domain', 'CertTemplate', 'Resource', 'AIAgent']
_EDGE_DEFS = [
    ('User', 'MemberOf', 'Group'),
    ('Group', 'MemberOf', 'Group'),
    ('User', 'AdminTo', 'Computer'),
    ('Group', 'AdminTo', 'Computer'),
    ('User', 'ActuallyAccessed', 'Resource'),
    ('Computer', 'LSASSAccessible', 'Computer'),
    ('User', 'CanEnrollInTemplate', 'CertTemplate'),
    ('AIAgent', 'AgentCanAccess', 'Resource'),
]
_EK = ['%s__%s__%s' % (s, r, d) for s, r, d in _EDGE_DEFS]


def _rup(x, m):
    return (x + m - 1) // m * m


def _mm_parts(x, ws, bs, bm=512):
    n, k = x.shape
    P = ws.shape[0]

    def body(x_ref, w_ref, b_ref, *out_refs):
        xb = x_ref[...]
        for p in range(P):
            out_refs[p][...] = (
                jnp.dot(xb, w_ref[p], preferred_element_type=jnp.float32,
                        precision=lax.Precision.HIGHEST)
                + b_ref[p][None, :])

    return pl.pallas_call(
        body,
        grid=(pl.cdiv(n, bm),),
        in_specs=[
            pl.BlockSpec((bm, k), lambda i: (i, 0)),
            pl.BlockSpec((P, k, 128), lambda i: (0, 0, 0)),
            pl.BlockSpec((P, 128), lambda i: (0, 0)),
        ],
        out_specs=[pl.BlockSpec((bm, 128), lambda i: (i, 0))] * P,
        out_shape=[jax.ShapeDtypeStruct((n, 128), jnp.float32)] * P,
    )(x, ws, bs)


def _comb_ln(accs, sss, mws, ow, ob, g, b, h_prev, bm=256):
    R = len(accs)
    np_rows = h_prev.shape[0]
    E = jnp.zeros((4, 128), jnp.float32)
    for hh in range(HEADS):
        E = E.at[hh, hh * HD:(hh + 1) * HD].set(1.0)

    def body(h_ref, mw_ref, ow_ref, ob_ref, g_ref, b_ref, e_ref, *refs):
        out_ref = refs[-1]
        acc_refs = refs[:R]
        ss_refs = refs[R:2 * R]
        y = jnp.zeros(h_ref.shape, jnp.float32)
        for r in range(R):
            num = acc_refs[r][0] + acc_refs[r][1]
            S = ss_refs[r][0] + ss_refs[r][1]
            den = jnp.dot(S, e_ref[...],
                          preferred_element_type=jnp.float32,
                        precision=lax.Precision.HIGHEST) + 1e-10
            y = y + jnp.dot(num / den, mw_ref[r],
                            preferred_element_type=jnp.float32,
                        precision=lax.Precision.HIGHEST)
        proj = jnp.dot(y, ow_ref[...], preferred_element_type=jnp.float32,
                        precision=lax.Precision.HIGHEST) \
            + ob_ref[...]
        t = h_ref[...] + proj
        m = t.mean(-1, keepdims=True)
        v = ((t - m) ** 2).mean(-1, keepdims=True)
        out_ref[...] = (t - m) / jnp.sqrt(v + 1e-5) * g_ref[...] + b_ref[...]

    return pl.pallas_call(
        body,
        grid=(pl.cdiv(np_rows, bm),),
        in_specs=[
            pl.BlockSpec((bm, 128), lambda i: (i, 0)),
            pl.BlockSpec((R, 128, 128), lambda i: (0, 0, 0)),
            pl.BlockSpec((128, 128), lambda i: (0, 0)),
            pl.BlockSpec((1, 128), lambda i: (0, 0)),
            pl.BlockSpec((1, 128), lambda i: (0, 0)),
            pl.BlockSpec((1, 128), lambda i: (0, 0)),
            pl.BlockSpec((4, 128), lambda i: (0, 0)),
        ] + [pl.BlockSpec((2, bm, 128), lambda i: (0, i, 0))] * R
          + [pl.BlockSpec((2, bm, 4), lambda i: (0, i, 0))] * R,
        out_specs=pl.BlockSpec((bm, 128), lambda i: (i, 0)),
        out_shape=jax.ShapeDtypeStruct((np_rows, 128), jnp.float32),
    )(h_prev, mws, ow, ob, g, b, E, *accs, *sss)


def _risk_mlp(h_all, w1, b1, w2, b2, bm=512):
    n = h_all.shape[0]

    def body(x_ref, w1_ref, b1_ref, w2_ref, b2_ref, o_ref):
        t = jnp.dot(x_ref[...], w1_ref[...],
                    preferred_element_type=jnp.float32,
                        precision=lax.Precision.HIGHEST) + b1_ref[...]
        t = jnp.where(t > 0, t, jnp.exp(t) - 1.0)
        r = jnp.dot(t, w2_ref[...], preferred_element_type=jnp.float32,
                        precision=lax.Precision.HIGHEST) \
            + b2_ref[...]
        o_ref[...] = 1.0 / (1.0 + jnp.exp(-r))

    return pl.pallas_call(
        body,
        grid=(pl.cdiv(n, bm),),
        in_specs=[
            pl.BlockSpec((bm, 128), lambda i: (i, 0)),
            pl.BlockSpec((128, 64), lambda i: (0, 0)),
            pl.BlockSpec((1, 64), lambda i: (0, 0)),
            pl.BlockSpec((64, 1), lambda i: (0, 0)),
            pl.BlockSpec((1, 1), lambda i: (0, 0)),
        ],
        out_specs=pl.BlockSpec((bm, 1), lambda i: (i, 0)),
        out_shape=jax.ShapeDtypeStruct((n, 1), jnp.float32),
    )(h_all, w1, b1, w2, b2)


def _sc_att(qt, kt, vt, si, di, ndacc):
    Ep = si.shape[0]
    eperw = Ep // _NW
    ngr = eperw // _CHUNK
    nd2 = ndacc // 32
    zrows = ndacc // _NS
    zfull = zrows // _CHUNK
    nblk2 = nd2 // 8
    mesh = plsc.VectorSubcoreMesh(core_axis_name="c", subcore_axis_name="s")

    @functools.partial(
        pl.kernel, mesh=mesh,
        compiler_params=pltpu.CompilerParams(needs_layout_passes=False),
        out_type=(jax.ShapeDtypeStruct((_NC, ndacc, 128), jnp.float32),
                  jax.ShapeDtypeStruct((_NC, nd2, 128), jnp.float32)),
        scratch_types=[
            pltpu.VMEM((_CHUNK,), jnp.int32),
            pltpu.VMEM((_CHUNK,), jnp.int32),
            pltpu.VMEM((_CHUNK,), jnp.int32),
            pltpu.VMEM((_CHUNK, 128), jnp.float32),
            pltpu.VMEM((_CHUNK, 128), jnp.float32),
            pltpu.VMEM((_CHUNK, 128), jnp.float32),
            pltpu.VMEM((_CHUNK, 128), jnp.float32),
            pltpu.VMEM((_CHUNK, 128), jnp.float32),
            pltpu.VMEM_SHARED((ndacc, 128), jnp.float32),
            pltpu.VMEM_SHARED((nd2, 128), jnp.float32),
            pltpu.SemaphoreType.DMA,
            pltpu.SemaphoreType.DMA,
            pltpu.SemaphoreType.DMA,
        ])
    def k(qt_h, kt_h, vt_h, si_h, di_h, out1_h, out2_h,
          si_v, di_v, di32_v, qv, kv, vv, wv1, wv2, acc1_sh, acc2_sh,
          s1, s2, s3):
        cid = lax.axis_index("c")
        sid = lax.axis_index("s")
        wid = sid * _NC + cid

        def zb(i, _):
            wv2[i // 8, pl.ds((i % 8) * 16, 16)] = jnp.zeros((16,),
                                                             jnp.float32)
            return 0
        lax.fori_loop(0, _CHUNK * 8, zb, 0, unroll=8)

        def zacc(j, _):
            pltpu.sync_copy(wv2, acc1_sh.at[pl.ds(sid * zrows + j * _CHUNK,
                                                  _CHUNK)])
            return 0
        lax.fori_loop(0, zfull, zacc, 0)
        for jj in range((nblk2 + _NS - 1) // _NS):
            blk = jj * _NS + sid

            @pl.when(blk < nblk2)
            def _():
                pltpu.sync_copy(wv2.at[pl.ds(0, 8)],
                                acc2_sh.at[pl.ds(blk * 8, 8)])
        plsc.subcore_barrier()

        def gbody(gi, _):
            base = wid * eperw + gi * _CHUNK
            pltpu.sync_copy(si_h.at[pl.ds(base, _CHUNK)], si_v)
            pltpu.sync_copy(di_h.at[pl.ds(base, _CHUNK)], di_v)
            c1 = pltpu.async_copy(qt_h.at[di_v], qv, s1)
            c2 = pltpu.async_copy(kt_h.at[si_v], kv, s2)
            c3 = pltpu.async_copy(vt_h.at[si_v], vv, s3)
            c1.wait()
            c2.wait()
            c3.wait()
            for g16 in range(_CHUNK // 16):
                rows = lax.iota(jnp.int32, 16) + (g16 * 16)
                di16 = di_v[pl.ds(g16 * 16, 16)]
                di32_v[pl.ds(g16 * 16, 16)] = di16 // 32
                colp = (di16 % 32) * 4
                for hh in range(HEADS):
                    col0 = jnp.full((16,), hh * HD, jnp.int32)

                    def dqk(dd, a):
                        col = col0 + dd
                        qd = plsc.load_gather(qv, [rows, col])
                        kd = plsc.load_gather(kv, [rows, col])
                        return a + qd * kd
                    att = lax.fori_loop(0, HD, dqk,
                                        jnp.zeros((16,), jnp.float32),
                                        unroll=4)
                    ex = jnp.exp(att)
                    plsc.store_scatter(wv2, [rows, colp + hh], ex)

                    def dv(dd, _):
                        col = col0 + dd
                        vd = plsc.load_gather(vv, [rows, col])
                        plsc.store_scatter(wv1, [rows, col], vd * ex)
                        return 0
                    lax.fori_loop(0, HD, dv, 0, unroll=4)
            pltpu.sync_copy(wv1, acc1_sh.at[di_v], add=True)
            pltpu.sync_copy(wv2, acc2_sh.at[di32_v], add=True)
            for g16 in range(_CHUNK // 16):
                rows = lax.iota(jnp.int32, 16) + (g16 * 16)
                di16 = di_v[pl.ds(g16 * 16, 16)]
                colp = (di16 % 32) * 4
                z16 = jnp.zeros((16,), jnp.float32)
                for hh in range(HEADS):
                    plsc.store_scatter(wv2, [rows, colp + hh], z16)
            return 0
        lax.fori_loop(0, ngr, gbody, 0)

        plsc.subcore_barrier()

        def wout(j, _):
            r0 = sid * zrows + j * _CHUNK
            pltpu.sync_copy(acc1_sh.at[pl.ds(r0, _CHUNK)],
                            out1_h.at[cid, pl.ds(r0, _CHUNK)])
            return 0
        lax.fori_loop(0, zfull, wout, 0)
        for jj in range((nblk2 + _NS - 1) // _NS):
            blk = jj * _NS + sid

            @pl.when(blk < nblk2)
            def _():
                pltpu.sync_copy(acc2_sh.at[pl.ds(blk * 8, 8)],
                                out2_h.at[cid, pl.ds(blk * 8, 8)])

    return k(qt, kt, vt, si, di)


def _sc_edge(at, bt, si, di, tab):
    Ep = si.shape[0]
    eperw = Ep // _NW
    ngr = eperw // _CHUNK
    mesh = plsc.VectorSubcoreMesh(core_axis_name="c", subcore_axis_name="s")

    @functools.partial(
        pl.kernel, mesh=mesh,
        compiler_params=pltpu.CompilerParams(needs_layout_passes=False),
        out_type=jax.ShapeDtypeStruct((Ep,), jnp.float32),
        scratch_types=[
            pltpu.VMEM((_CHUNK,), jnp.int32),
            pltpu.VMEM((_CHUNK,), jnp.int32),
            pltpu.VMEM((_CHUNK, 128), jnp.float32),
            pltpu.VMEM((_CHUNK, 128), jnp.float32),
            pltpu.VMEM((_CHUNK,), jnp.float32),
            pltpu.VMEM((264, 16), jnp.float32),
            pltpu.SemaphoreType.DMA,
            pltpu.SemaphoreType.DMA,
        ])
    def k(at_h, bt_h, si_h, di_h, tab_h, out_h,
          si_v, di_v, av, bv, ov, tab_v, s1, s2):
        cid = lax.axis_index("c")
        sid = lax.axis_index("s")
        wid = sid * _NC + cid
        pltpu.sync_copy(tab_h, tab_v)

        def gbody(gi, _):
            base = wid * eperw + gi * _CHUNK
            pltpu.sync_copy(si_h.at[pl.ds(base, _CHUNK)], si_v)
            pltpu.sync_copy(di_h.at[pl.ds(base, _CHUNK)], di_v)
            c1 = pltpu.async_copy(at_h.at[si_v], av, s1)
            c2 = pltpu.async_copy(bt_h.at[di_v], bv, s2)
            c1.wait()
            c2.wait()
            for g16 in range(_CHUNK // 16):
                rows = lax.iota(jnp.int32, 16) + (g16 * 16)

                def dd(d, a):
                    ad = plsc.load_gather(av, [rows, jnp.full((16,), 0,
                                                              jnp.int32) + d])
                    bd = plsc.load_gather(bv, [rows, jnp.full((16,), 0,
                                                              jnp.int32) + d])
                    z = ad + bd + tab_v[d]
                    z = jnp.where(z > 0, z, jnp.exp(z) - 1.0)
                    return a + z * tab_v[128 + d]
                acc = lax.fori_loop(0, 128, dd,
                                    jnp.zeros((16,), jnp.float32), unroll=4)
                logit = acc + tab_v[256]
                ov[pl.ds(g16 * 16, 16)] = 1.0 / (1.0 + jnp.exp(-logit))
            pltpu.sync_copy(ov, out_h.at[pl.ds(base, _CHUNK)])
            return 0
        lax.fori_loop(0, ngr, gbody, 0)

    return k(at, bt, si, di, tab)


def kernel(x_User, x_Computer, x_Group, x_Domain, x_CertTemplate, x_Resource,
           x_AIAgent, ei_0, ei_1, ei_2, ei_3, ei_4, ei_5, ei_6, ei_7,
           theta, params):
    xs = {'User': x_User, 'Computer': x_Computer, 'Group': x_Group,
          'Domain': x_Domain, 'CertTemplate': x_CertTemplate,
          'Resource': x_Resource, 'AIAgent': x_AIAgent}
    eis = [ei_0, ei_1, ei_2, ei_3, ei_4, ei_5, ei_6, ei_7]
    counts = {nt: xs[nt].shape[0] for nt in _NTS}
    nprows = {nt: _rup(counts[nt] + 1, 8) for nt in _NTS}
    scale = HD ** -0.5

    h = {}
    for nt in _NTS:
        x = xs[nt]
        xp = jnp.pad(x, ((0, nprows[nt] - x.shape[0]), (0, 32 - x.shape[1])))
        wp = jnp.pad(params['inproj'][nt]['w'],
                     ((0, 32 - x.shape[1]), (0, 0)))
        h[nt] = _mm_parts(xp, wp[None], params['inproj'][nt]['b'][None])[0]

    sis, dis, eps = [], [], []
    for i, (s, r, d) in enumerate(_EDGE_DEFS):
        e = eis[i].shape[1]
        ep = _rup(e, _EALIGN)
        si = jnp.pad(eis[i][0].astype(jnp.int32), (0, ep - e))
        di = jnp.pad(eis[i][1].astype(jnp.int32), (0, ep - e),
                     constant_values=counts[d])
        sis.append(si)
        dis.append(di)
        eps.append(ep)
    ndaccs = [_rup(counts[d] + 1, 512) for (_, _, d) in _EDGE_DEFS]

    dst_rels = {}
    for i, (s, r, d) in enumerate(_EDGE_DEFS):
        dst_rels.setdefault(d, []).append(i)

    for lay in params['layers']:
        jobs = {nt: [] for nt in _NTS}
        for i, (s, r, d) in enumerate(_EDGE_DEFS):
            jobs[s].append(('k', i))
            jobs[s].append(('v', i))
            jobs[d].append(('q', i))
        tabs = {}
        for nt in _NTS:
            if not jobs[nt]:
                continue
            wlist = []
            for role, i in jobs[nt]:
                w = lay[role][_EK[i]]
                if role == 'q':
                    w = w * (scale * jax.nn.sigmoid(theta[i]))
                wlist.append(w)
            ws = jnp.stack(wlist)
            bs = jnp.zeros((len(wlist), 128), jnp.float32)
            outs = _mm_parts(h[nt], ws, bs)
            for (role, i), o in zip(jobs[nt], outs):
                tabs[(role, i)] = o

        accs, sss = [], []
        for i, (s, r, d) in enumerate(_EDGE_DEFS):
            a1, a2 = _sc_att(tabs[('q', i)], tabs[('k', i)],
                             tabs[('v', i)], sis[i], dis[i], ndaccs[i])
            accs.append(a1)
            sss.append(a2.reshape(_NC, ndaccs[i], 4))

        for nt, rels in dst_rels.items():
            rws = jnp.stack([jax.nn.sigmoid(lay['imp'][_EK[i]])
                             for i in rels])
            c = jax.nn.softmax(rws) * rws
            mws = jnp.stack([lay['mw'][_EK[i]] * c[j]
                             for j, i in enumerate(rels)])
            h[nt] = _comb_ln([accs[i] for i in rels],
                             [sss[i] for i in rels], mws, lay['ow'][nt],
                             lay['ob'][nt][None], lay['lg'][nt][None],
                             lay['lb'][nt][None], h[nt])

    h_all = jnp.concatenate([h[nt] for nt in _NTS], axis=0)
    offs = {}
    o = 0
    for nt in _NTS:
        offs[nt] = o
        o += nprows[nt]

    rp = params['risk']
    risk_all = _risk_mlp(h_all, rp['w1'], rp['b1'][None], rp['w2'],
                         rp['b2'][None])
    risk = {nt: risk_all[offs[nt]:offs[nt] + counts[nt]] for nt in _NTS}

    ep = params['edge']
    ab = _mm_parts(h_all, jnp.stack([ep['w1'][:H], ep['w1'][H:]]),
                   jnp.zeros((2, 128), jnp.float32))
    tab = jnp.zeros((264, 16), jnp.float32)
    tab = tab.at[0:128].set(jnp.broadcast_to(ep['b1'][:, None], (128, 16)))
    tab = tab.at[128:256].set(jnp.broadcast_to(ep['w2'], (128, 16)))
    tab = tab.at[256].set(ep['b2'][0])

    gsis, gdis, eoffs = [], [], []
    eo = 0
    for i, (s, r, d) in enumerate(_EDGE_DEFS):
        gsis.append(sis[i] + offs[s])
        gdis.append(jnp.minimum(dis[i], counts[d] - 1) + offs[d])
        eoffs.append(eo)
        eo += eps[i]
    gsi = jnp.concatenate(gsis)
    gdi = jnp.concatenate(gdis)
    eprob_all = _sc_edge(ab[0], ab[1], gsi, gdi, tab)
    eprob = {}
    for i in range(8):
        e = eis[i].shape[1]
        eprob[_EK[i]] = eprob_all[eoffs[i]:eoffs[i] + e][:, None]

    return (risk, eprob)


# speedup vs baseline: 2.2127x; 1.9152x over previous
import functools

import jax
import jax.numpy as jnp
from jax import lax
from jax.experimental import pallas as pl
from jax.experimental.pallas import tpu as pltpu
from jax.experimental.pallas import tpu_sc as plsc

H = 128
HEADS = 4
HD = H // HEADS
_NC, _NS = 2, 16
_NW = _NC * _NS
_CHUNK = 64
_EALIGN = _NW * _CHUNK

_NTS = ['User', 'Computer', 'Group', 'Domain', 'CertTemplate', 'Resource', 'AIAgent']
_EDGE_DEFS = [
    ('User', 'MemberOf', 'Group'),
    ('Group', 'MemberOf', 'Group'),
    ('User', 'AdminTo', 'Computer'),
    ('Group', 'AdminTo', 'Computer'),
    ('User', 'ActuallyAccessed', 'Resource'),
    ('Computer', 'LSASSAccessible', 'Computer'),
    ('User', 'CanEnrollInTemplate', 'CertTemplate'),
    ('AIAgent', 'AgentCanAccess', 'Resource'),
]
_EK = ['%s__%s__%s' % (s, r, d) for s, r, d in _EDGE_DEFS]


def _rup(x, m):
    return (x + m - 1) // m * m


def _mm_parts(x, ws, bs, bm=512):
    n, k = x.shape
    P, _, W = ws.shape

    def body(x_ref, w_ref, b_ref, *out_refs):
        xb = x_ref[...]
        for p in range(P):
            out_refs[p][...] = (
                jnp.dot(xb, w_ref[p], preferred_element_type=jnp.float32,
                        precision=lax.Precision.HIGHEST)
                + b_ref[p][None, :])

    return pl.pallas_call(
        body,
        grid=(pl.cdiv(n, bm),),
        in_specs=[
            pl.BlockSpec((bm, k), lambda i: (i, 0)),
            pl.BlockSpec((P, k, W), lambda i: (0, 0, 0)),
            pl.BlockSpec((P, W), lambda i: (0, 0)),
        ],
        out_specs=[pl.BlockSpec((bm, W), lambda i: (i, 0))] * P,
        out_shape=[jax.ShapeDtypeStruct((n, W), jnp.float32)] * P,
    )(x, ws, bs)


def _comb_ln(accs, sss, mws, ow, ob, g, b, h_prev, bm=256):
    R = len(accs)
    np_rows = h_prev.shape[0]
    E = jnp.zeros((4, 128), jnp.float32)
    for hh in range(HEADS):
        E = E.at[hh, hh * HD:(hh + 1) * HD].set(1.0)

    def body(h_ref, mw_ref, ow_ref, ob_ref, g_ref, b_ref, e_ref, *refs):
        out_ref = refs[-1]
        acc_refs = refs[:R]
        ss_refs = refs[R:2 * R]
        y = jnp.zeros(h_ref.shape, jnp.float32)
        for r in range(R):
            num = acc_refs[r][0] + acc_refs[r][1]
            S = ss_refs[r][0] + ss_refs[r][1]
            den = jnp.dot(S, e_ref[...],
                          preferred_element_type=jnp.float32,
                        precision=lax.Precision.HIGHEST) + 1e-10
            y = y + jnp.dot(num / den, mw_ref[r],
                            preferred_element_type=jnp.float32,
                        precision=lax.Precision.HIGHEST)
        proj = jnp.dot(y, ow_ref[...], preferred_element_type=jnp.float32,
                        precision=lax.Precision.HIGHEST) \
            + ob_ref[...]
        t = h_ref[...] + proj
        m = t.mean(-1, keepdims=True)
        v = ((t - m) ** 2).mean(-1, keepdims=True)
        out_ref[...] = (t - m) / jnp.sqrt(v + 1e-5) * g_ref[...] + b_ref[...]

    return pl.pallas_call(
        body,
        grid=(pl.cdiv(np_rows, bm),),
        in_specs=[
            pl.BlockSpec((bm, 128), lambda i: (i, 0)),
            pl.BlockSpec((R, 128, 128), lambda i: (0, 0, 0)),
            pl.BlockSpec((128, 128), lambda i: (0, 0)),
            pl.BlockSpec((1, 128), lambda i: (0, 0)),
            pl.BlockSpec((1, 128), lambda i: (0, 0)),
            pl.BlockSpec((1, 128), lambda i: (0, 0)),
            pl.BlockSpec((4, 128), lambda i: (0, 0)),
        ] + [pl.BlockSpec((2, bm, 128), lambda i: (0, i, 0))] * R
          + [pl.BlockSpec((2, bm, 4), lambda i: (0, i, 0))] * R,
        out_specs=pl.BlockSpec((bm, 128), lambda i: (i, 0)),
        out_shape=jax.ShapeDtypeStruct((np_rows, 128), jnp.float32),
    )(h_prev, mws, ow, ob, g, b, E, *accs, *sss)


def _risk_mlp(h_all, w1, b1, w2, b2, bm=512):
    n = h_all.shape[0]

    def body(x_ref, w1_ref, b1_ref, w2_ref, b2_ref, o_ref):
        t = jnp.dot(x_ref[...], w1_ref[...],
                    preferred_element_type=jnp.float32,
                        precision=lax.Precision.HIGHEST) + b1_ref[...]
        t = jnp.where(t > 0, t, jnp.exp(t) - 1.0)
        r = jnp.dot(t, w2_ref[...], preferred_element_type=jnp.float32,
                        precision=lax.Precision.HIGHEST) \
            + b2_ref[...]
        o_ref[...] = 1.0 / (1.0 + jnp.exp(-r))

    return pl.pallas_call(
        body,
        grid=(pl.cdiv(n, bm),),
        in_specs=[
            pl.BlockSpec((bm, 128), lambda i: (i, 0)),
            pl.BlockSpec((128, 64), lambda i: (0, 0)),
            pl.BlockSpec((1, 64), lambda i: (0, 0)),
            pl.BlockSpec((64, 1), lambda i: (0, 0)),
            pl.BlockSpec((1, 1), lambda i: (0, 0)),
        ],
        out_specs=pl.BlockSpec((bm, 1), lambda i: (i, 0)),
        out_shape=jax.ShapeDtypeStruct((n, 1), jnp.float32),
    )(h_all, w1, b1, w2, b2)


def _sc_att(qt, kvt, si, di, ndacc, tok):
    Ep = si.shape[0]
    eperw = Ep // _NW
    ngr = eperw // _CHUNK
    npairs = (ngr + 1) // 2
    nd2 = ndacc // 32
    zrows = ndacc // _NS
    zfull, zrem = zrows // _CHUNK, zrows % _CHUNK
    nblk2 = nd2 // 8
    mesh = plsc.VectorSubcoreMesh(core_axis_name="c", subcore_axis_name="s")

    @functools.partial(
        pl.kernel, mesh=mesh,
        compiler_params=pltpu.CompilerParams(needs_layout_passes=False),
        out_type=(jax.ShapeDtypeStruct((_NC, ndacc, 128), jnp.float32),
                  jax.ShapeDtypeStruct((_NC, nd2, 128), jnp.float32)),
        scratch_types=[
            pltpu.VMEM((_CHUNK,), jnp.int32),
            pltpu.VMEM((_CHUNK,), jnp.int32),
            pltpu.VMEM((_CHUNK,), jnp.int32),
            pltpu.VMEM((_CHUNK,), jnp.int32),
            pltpu.VMEM((_CHUNK,), jnp.int32),
            pltpu.VMEM((_CHUNK, 128), jnp.float32),
            pltpu.VMEM((_CHUNK, 128), jnp.float32),
            pltpu.VMEM((_CHUNK, 256), jnp.float32),
            pltpu.VMEM((_CHUNK, 256), jnp.float32),
            pltpu.VMEM((_CHUNK, 128), jnp.float32),
            pltpu.VMEM((_CHUNK, 128), jnp.float32),
            pltpu.VMEM((8,), jnp.float32),
            pltpu.VMEM_SHARED((ndacc, 128), jnp.float32),
            pltpu.VMEM_SHARED((nd2, 128), jnp.float32),
            pltpu.SemaphoreType.DMA,
            pltpu.SemaphoreType.DMA,
            pltpu.SemaphoreType.DMA,
            pltpu.SemaphoreType.DMA,
        ])
    def k(qt_h, kvt_h, si_h, di_h, tok_h, out1_h, out2_h,
          si0_v, si1_v, di0_v, di1_v, di32_v, qv0, qv1, kv0, kv1,
          wv1, wv2, tok_v, acc1_sh, acc2_sh, sq0, sq1, skv0, skv1):
        pltpu.sync_copy(tok_h, tok_v)
        cid = lax.axis_index("c")
        sid = lax.axis_index("s")
        wid = sid * _NC + cid

        def zb(i, _):
            wv2[i // 8, pl.ds((i % 8) * 16, 16)] = jnp.zeros((16,),
                                                             jnp.float32)
            return 0
        lax.fori_loop(0, _CHUNK * 8, zb, 0, unroll=8)

        def zacc(j, _):
            pltpu.sync_copy(wv2, acc1_sh.at[pl.ds(sid * zrows + j * _CHUNK,
                                                  _CHUNK)])
            return 0
        lax.fori_loop(0, zfull, zacc, 0)
        if zrem:
            pltpu.sync_copy(wv2.at[pl.ds(0, zrem)],
                            acc1_sh.at[pl.ds(sid * zrows + zfull * _CHUNK,
                                             zrem)])
        for jj in range((nblk2 + _NS - 1) // _NS):
            blk = jj * _NS + sid

            @pl.when(blk < nblk2)
            def _():
                pltpu.sync_copy(wv2.at[pl.ds(0, 8)],
                                acc2_sh.at[pl.ds(blk * 8, 8)])
        plsc.subcore_barrier()

        def issue(sib, dib, qvb, kvb, sqb, skvb, gi):
            base = wid * eperw + gi * _CHUNK
            pltpu.sync_copy(si_h.at[pl.ds(base, _CHUNK)], sib)
            pltpu.sync_copy(di_h.at[pl.ds(base, _CHUNK)], dib)
            pltpu.async_copy(qt_h.at[dib], qvb, sqb)
            pltpu.async_copy(kvt_h.at[sib], kvb, skvb)

        lane0 = lax.iota(jnp.int32, 16) == 0

        def compute(sib, dib, qvb, kvb, sqb, skvb):
            pltpu.make_async_copy(qt_h.at[dib], qvb, sqb).wait()
            pltpu.make_async_copy(kvt_h.at[sib], kvb, skvb).wait()
            for g16 in range(_CHUNK // 16):
                di16 = dib[pl.ds(g16 * 16, 16)]
                di32_v[pl.ds(g16 * 16, 16)] = di16 // 32

            def ebody(e, _):
                qk = [qvb[e, pl.ds(j * 16, 16)] * kvb[e, pl.ds(j * 16, 16)]
                      for j in range(8)]
                dsp = plsc.load_gather(dib, [jnp.full((16,), e, jnp.int32)])
                colv = (dsp % 32) * 4
                erow = jnp.full((16,), e, jnp.int32)
                for hh in range(HEADS):
                    att = jnp.sum(qk[2 * hh] + qk[2 * hh + 1])
                    ex = jnp.exp(jnp.full((16,), att, jnp.float32))
                    wv1[e, pl.ds(hh * 32, 16)] = \
                        kvb[e, pl.ds(128 + hh * 32, 16)] * ex
                    wv1[e, pl.ds(hh * 32 + 16, 16)] = \
                        kvb[e, pl.ds(144 + hh * 32, 16)] * ex
                    plsc.store_scatter(wv2, [erow, colv + hh], ex,
                                       mask=lane0)
                return 0
            lax.fori_loop(0, _CHUNK, ebody, 0, unroll=2)
            pltpu.sync_copy(wv1, acc1_sh.at[dib], add=True)
            pltpu.sync_copy(wv2, acc2_sh.at[di32_v], add=True)
            def zclr(e, _):
                dsp = plsc.load_gather(dib, [jnp.full((16,), e, jnp.int32)])
                colv = (dsp % 32) * 4
                erow = jnp.full((16,), e, jnp.int32)
                z16 = jnp.zeros((16,), jnp.float32)
                for hh in range(HEADS):
                    plsc.store_scatter(wv2, [erow, colv + hh], z16,
                                       mask=lane0)
                return 0
            lax.fori_loop(0, _CHUNK, zclr, 0, unroll=2)

        def gbody(gi, _):
            issue(si0_v, di0_v, qv0, kv0, sq0, skv0, gi)
            compute(si0_v, di0_v, qv0, kv0, sq0, skv0)
            return 0
        lax.fori_loop(0, ngr, gbody, 0)

        plsc.subcore_barrier()

        def wout(j, _):
            r0 = sid * zrows + j * _CHUNK
            pltpu.sync_copy(acc1_sh.at[pl.ds(r0, _CHUNK)],
                            out1_h.at[cid, pl.ds(r0, _CHUNK)])
            return 0
        lax.fori_loop(0, zfull, wout, 0)
        if zrem:
            r0 = sid * zrows + zfull * _CHUNK
            pltpu.sync_copy(acc1_sh.at[pl.ds(r0, zrem)],
                            out1_h.at[cid, pl.ds(r0, zrem)])
        for jj in range((nblk2 + _NS - 1) // _NS):
            blk = jj * _NS + sid

            @pl.when(blk < nblk2)
            def _():
                pltpu.sync_copy(acc2_sh.at[pl.ds(blk * 8, 8)],
                                out2_h.at[cid, pl.ds(blk * 8, 8)])

    return k(qt, kvt, si, di, tok)


def _sc_edge(at, bt, si, di, tab):
    Ep = si.shape[0]
    eperw = Ep // _NW
    ngr = eperw // _CHUNK
    npairs = (ngr + 1) // 2
    mesh = plsc.VectorSubcoreMesh(core_axis_name="c", subcore_axis_name="s")

    @functools.partial(
        pl.kernel, mesh=mesh,
        compiler_params=pltpu.CompilerParams(needs_layout_passes=False),
        out_type=jax.ShapeDtypeStruct((Ep,), jnp.float32),
        scratch_types=[
            pltpu.VMEM((_CHUNK,), jnp.int32),
            pltpu.VMEM((_CHUNK,), jnp.int32),
            pltpu.VMEM((_CHUNK,), jnp.int32),
            pltpu.VMEM((_CHUNK,), jnp.int32),
            pltpu.VMEM((_CHUNK, 128), jnp.float32),
            pltpu.VMEM((_CHUNK, 128), jnp.float32),
            pltpu.VMEM((_CHUNK, 128), jnp.float32),
            pltpu.VMEM((_CHUNK, 128), jnp.float32),
            pltpu.VMEM((_CHUNK,), jnp.float32),
            pltpu.VMEM((3, 128), jnp.float32),
            pltpu.SemaphoreType.DMA,
            pltpu.SemaphoreType.DMA,
            pltpu.SemaphoreType.DMA,
            pltpu.SemaphoreType.DMA,
        ])
    def k(at_h, bt_h, si_h, di_h, tab_h, out_h,
          si0_v, si1_v, di0_v, di1_v, av0, av1, bv0, bv1, ov, tab_v,
          sa0, sa1, sb0, sb1):
        cid = lax.axis_index("c")
        sid = lax.axis_index("s")
        wid = sid * _NC + cid
        pltpu.sync_copy(tab_h, tab_v)
        lane0 = lax.iota(jnp.int32, 16) == 0
        b1s = [tab_v[0, pl.ds(j * 16, 16)] for j in range(8)]
        w2s = [tab_v[1, pl.ds(j * 16, 16)] for j in range(8)]
        b2s = tab_v[2, pl.ds(0, 16)]

        def issue(sib, dib, avb, bvb, sab, sbb, gi):
            base = wid * eperw + gi * _CHUNK
            pltpu.sync_copy(si_h.at[pl.ds(base, _CHUNK)], sib)
            pltpu.sync_copy(di_h.at[pl.ds(base, _CHUNK)], dib)
            pltpu.async_copy(at_h.at[sib], avb, sab)
            pltpu.async_copy(bt_h.at[dib], bvb, sbb)

        def compute(sib, dib, avb, bvb, sab, sbb, gi):
            base = wid * eperw + gi * _CHUNK
            pltpu.make_async_copy(at_h.at[sib], avb, sab).wait()
            pltpu.make_async_copy(bt_h.at[dib], bvb, sbb).wait()

            def ebody(e, _):
                acc = jnp.zeros((16,), jnp.float32)
                for j in range(8):
                    z = (avb[e, pl.ds(j * 16, 16)]
                         + bvb[e, pl.ds(j * 16, 16)] + b1s[j])
                    z = jnp.where(z > 0, z, jnp.exp(z) - 1.0)
                    acc = acc + z * w2s[j]
                s = jnp.sum(acc)
                logit = jnp.full((16,), s, jnp.float32) + b2s
                sig = 1.0 / (1.0 + jnp.exp(-logit))
                plsc.store_scatter(ov, [jnp.full((16,), e, jnp.int32)],
                                   sig, mask=lane0)
                return 0
            lax.fori_loop(0, _CHUNK, ebody, 0, unroll=2)
            pltpu.sync_copy(ov, out_h.at[pl.ds(base, _CHUNK)])

        issue(si0_v, di0_v, av0, bv0, sa0, sb0, 0)

        def pair(p, _):
            g0 = 2 * p

            @pl.when(g0 + 1 < ngr)
            def _():
                issue(si1_v, di1_v, av1, bv1, sa1, sb1, g0 + 1)
            compute(si0_v, di0_v, av0, bv0, sa0, sb0, g0)

            @pl.when(g0 + 2 < ngr)
            def _():
                issue(si0_v, di0_v, av0, bv0, sa0, sb0, g0 + 2)

            @pl.when(g0 + 1 < ngr)
            def _():
                compute(si1_v, di1_v, av1, bv1, sa1, sb1, g0 + 1)
            return 0
        lax.fori_loop(0, npairs, pair, 0)

    return k(at, bt, si, di, tab)


def kernel(x_User, x_Computer, x_Group, x_Domain, x_CertTemplate, x_Resource,
           x_AIAgent, ei_0, ei_1, ei_2, ei_3, ei_4, ei_5, ei_6, ei_7,
           theta, params):
    xs = {'User': x_User, 'Computer': x_Computer, 'Group': x_Group,
          'Domain': x_Domain, 'CertTemplate': x_CertTemplate,
          'Resource': x_Resource, 'AIAgent': x_AIAgent}
    eis = [ei_0, ei_1, ei_2, ei_3, ei_4, ei_5, ei_6, ei_7]
    counts = {nt: xs[nt].shape[0] for nt in _NTS}
    nprows = {nt: _rup(counts[nt] + 1, 8) for nt in _NTS}
    scale = HD ** -0.5

    h = {}
    for nt in _NTS:
        x = xs[nt]
        xp = jnp.pad(x, ((0, nprows[nt] - x.shape[0]), (0, 32 - x.shape[1])))
        wp = jnp.pad(params['inproj'][nt]['w'],
                     ((0, 32 - x.shape[1]), (0, 0)))
        h[nt] = _mm_parts(xp, wp[None], params['inproj'][nt]['b'][None])[0]

    sis, dis, eps = [], [], []
    for i, (s, r, d) in enumerate(_EDGE_DEFS):
        e = eis[i].shape[1]
        ep = _rup(e, _EALIGN)
        si = jnp.pad(eis[i][0].astype(jnp.int32), (0, ep - e))
        di = jnp.pad(eis[i][1].astype(jnp.int32), (0, ep - e),
                     constant_values=counts[d])
        sis.append(si)
        dis.append(di)
        eps.append(ep)
    ndaccs = [_rup(counts[d] + 1, 512) for (_, _, d) in _EDGE_DEFS]

    dst_rels = {}
    for i, (s, r, d) in enumerate(_EDGE_DEFS):
        dst_rels.setdefault(d, []).append(i)

    tok = jnp.zeros((8,), jnp.float32)
    for lay in params['layers']:
        qjobs = {nt: [] for nt in _NTS}
        kvjobs = {nt: [] for nt in _NTS}
        for i, (s, r, d) in enumerate(_EDGE_DEFS):
            kvjobs[s].append(i)
            qjobs[d].append(i)
        qtabs, kvtabs = {}, {}
        for nt in _NTS:
            if qjobs[nt]:
                ws = jnp.stack([lay['q'][_EK[i]]
                                * (scale * jax.nn.sigmoid(theta[i]))
                                for i in qjobs[nt]])
                bs = jnp.zeros((len(qjobs[nt]), 128), jnp.float32)
                for i, o in zip(qjobs[nt], _mm_parts(h[nt], ws, bs)):
                    qtabs[i] = o
            if kvjobs[nt]:
                ws = jnp.stack([jnp.concatenate(
                    [lay['k'][_EK[i]], lay['v'][_EK[i]]], axis=1)
                    for i in kvjobs[nt]])
                bs = jnp.zeros((len(kvjobs[nt]), 256), jnp.float32)
                for i, o in zip(kvjobs[nt], _mm_parts(h[nt], ws, bs)):
                    kvtabs[i] = o

        accs, sss = [], []
        for i, (s, r, d) in enumerate(_EDGE_DEFS):
            a1, a2 = _sc_att(qtabs[i], kvtabs[i], sis[i], dis[i], ndaccs[i],
                             tok)
            tok = a2[0, 0, :8]
            accs.append(a1)
            sss.append(a2.reshape(_NC, ndaccs[i], 4))

        for nt, rels in dst_rels.items():
            rws = jnp.stack([jax.nn.sigmoid(lay['imp'][_EK[i]])
                             for i in rels])
            c = jax.nn.softmax(rws) * rws
            mws = jnp.stack([lay['mw'][_EK[i]] * c[j]
                             for j, i in enumerate(rels)])
            h[nt] = _comb_ln([accs[i] for i in rels],
                             [sss[i] for i in rels], mws, lay['ow'][nt],
                             lay['ob'][nt][None], lay['lg'][nt][None],
                             lay['lb'][nt][None], h[nt])

    h_all = jnp.concatenate([h[nt] for nt in _NTS], axis=0)
    offs = {}
    o = 0
    for nt in _NTS:
        offs[nt] = o
        o += nprows[nt]

    rp = params['risk']
    risk_all = _risk_mlp(h_all, rp['w1'], rp['b1'][None], rp['w2'],
                         rp['b2'][None])
    risk = {nt: risk_all[offs[nt]:offs[nt] + counts[nt]] for nt in _NTS}

    ep = params['edge']
    ab = _mm_parts(h_all, jnp.stack([ep['w1'][:H], ep['w1'][H:]]),
                   jnp.zeros((2, 128), jnp.float32))
    tab = jnp.stack([ep['b1'], ep['w2'][:, 0],
                     jnp.broadcast_to(ep['b2'], (128,))])

    gsis, gdis, eoffs = [], [], []
    eo = 0
    for i, (s, r, d) in enumerate(_EDGE_DEFS):
        gsis.append(sis[i] + offs[s])
        gdis.append(jnp.minimum(dis[i], counts[d] - 1) + offs[d])
        eoffs.append(eo)
        eo += eps[i]
    gsi = jnp.concatenate(gsis)
    gdi = jnp.concatenate(gdis)
    eprob_all = _sc_edge(ab[0], ab[1], gsi, gdi, tab)
    eprob = {}
    for i in range(8):
        e = eis[i].shape[1]
        eprob[_EK[i]] = eprob_all[eoffs[i]:eoffs[i] + e][:, None]

    return (risk, eprob)
